# baseline (speedup 1.0000x reference)
import functools

import jax
import jax.numpy as jnp
from jax.experimental import pallas as pl

NUM_QUAY = 64
NUM_OPS = 50000
N_SHIPS = 1024

_POOL_TILE = 1000
_SHIP_TILE = 64


def _elu(x):
    return jnp.where(x > 0, x, jnp.exp(jnp.minimum(x, 0.0)) - 1.0)


def _dot_hi(a, b):
    return jnp.dot(a, b, preferred_element_type=jnp.float32,
                   precision=jax.lax.Precision.HIGHEST)


def _quay_kernel(xq, Wq0, bq0, Wq1, bq1, Wa0q, ba0, qterm_ref, qpool_ref):
    h = _elu(_dot_hi(xq[...], Wq0[...]) + bq0[...])
    h = _elu(_dot_hi(h, Wq1[...]) + bq1[...])
    qterm_ref[...] = _dot_hi(h, Wa0q[...]) + ba0[...]
    qpool_ref[...] = jnp.mean(h, axis=0, keepdims=True)


def _pool_kernel(xo, Wo0, bo0, Wo1, bo1, acc_ref):
    h = _elu(_dot_hi(xo[...], Wo0[...]) + bo0[...])
    h = _elu(_dot_hi(h, Wo1[...]) + bo1[...])
    part = jnp.sum(h, axis=0, keepdims=True)

    @pl.when(pl.program_id(0) == 0)
    def _init():
        acc_ref[...] = part

    @pl.when(pl.program_id(0) != 0)
    def _acc():
        acc_ref[...] += part


def _actor_kernel(xg, ai, qterm, Wo0, bo0, Wo1, bo1, Wf0, bf0, Wf1, bf1,
                  Wa0s, Wa0f, Wa1, ba1, Wa2, ba2, out_ref):
    S = xg.shape[0]
    B = S * NUM_QUAY
    h = _elu(jnp.dot(xg[...], Wo0[...], preferred_element_type=jnp.float32) + bo0[...])
    h_ship = _elu(jnp.dot(h, Wo1[...], preferred_element_type=jnp.float32) + bo1[...])
    s_term = jnp.dot(h_ship, Wa0s[...], preferred_element_type=jnp.float32)
    a1 = jnp.dot(ai[...], Wf0[...], preferred_element_type=jnp.float32) + bf0[...]
    h_add = _elu(jnp.dot(_elu(a1), Wf1[...], preferred_element_type=jnp.float32) + bf1[...])
    act = jnp.dot(h_add, Wa0f[...], preferred_element_type=jnp.float32)
    act = act.reshape(S, NUM_QUAY, 256) + s_term[:, None, :] + qterm[...][None, :, :]
    act = _elu(act).reshape(B, 256)
    h2 = _elu(jnp.dot(act, Wa1[...], preferred_element_type=jnp.float32) + ba1[...])
    logit = jnp.dot(h2, Wa2[...], preferred_element_type=jnp.float32) + ba2[...]
    out_ref[...] = logit.reshape(S, NUM_QUAY)


def _final_kernel(logits, mask_t, qpool, opsum, Wc0q, Wc0o, bc0, Wc1, bc1,
                  Wc2, bc2, act_ref, lp_ref, sv_ref):
    lg = jnp.where(mask_t[...], logits[...], -jnp.inf)
    m = jnp.max(lg)
    ids = (jax.lax.broadcasted_iota(jnp.int32, lg.shape, 0) * NUM_QUAY
           + jax.lax.broadcasted_iota(jnp.int32, lg.shape, 1))
    amax = jnp.min(jnp.where(lg == m, ids, jnp.int32(2**31 - 1)))
    lse = m + jnp.log(jnp.sum(jnp.exp(lg - m)))
    act_ref[...] = amax.reshape(1, 1)
    lp_ref[...] = (m - lse).reshape(1, 1)
    op = opsum[...] * (1.0 / NUM_OPS)
    hc = _elu(_dot_hi(qpool[...], Wc0q[...])
              + _dot_hi(op, Wc0o[...])
              + bc0[...])
    hc = _elu(_dot_hi(hc, Wc1[...]) + bc1[...])
    sv_ref[...] = _dot_hi(hc, Wc2[...]) + bc2[...]


def _row(x):
    return x.reshape(1, -1)


def kernel(x_quay, x_operation, mask, current_ops, added_info, Wq0, bq0, Wq1,
           bq1, Wo0, bo0, Wo1, bo1, Wf0, bf0, Wf1, bf1, Wa0, ba0, Wa1, ba1,
           Wa2, ba2, Wc0, bc0, Wc1, bc1, Wc2, bc2):
    f32 = jnp.float32
    D = 256
    Wa0q, Wa0s, Wa0f = Wa0[:D], Wa0[D:2 * D], Wa0[2 * D:]
    Wc0q, Wc0o = Wc0[:D], Wc0[D:]
    ai = added_info.reshape(N_SHIPS * NUM_QUAY, 2)
    mask_t = jnp.transpose(mask)
    xg = jnp.take(x_operation, current_ops, axis=0)

    full = lambda shape: pl.BlockSpec(shape, lambda *_: tuple(0 for _ in shape))

    qterm, qpool = pl.pallas_call(
        _quay_kernel,
        out_shape=(jax.ShapeDtypeStruct((NUM_QUAY, D), f32),
                   jax.ShapeDtypeStruct((1, D), f32)),
    )(x_quay, Wq0, _row(bq0), Wq1, _row(bq1), Wa0q, _row(ba0))

    n_pool = NUM_OPS // _POOL_TILE
    opsum = pl.pallas_call(
        _pool_kernel,
        grid=(n_pool,),
        in_specs=[
            pl.BlockSpec((_POOL_TILE, 128), lambda i: (i, 0)),
            full((128, D)), full((1, D)), full((D, D)), full((1, D)),
        ],
        out_specs=pl.BlockSpec((1, D), lambda i: (0, 0)),
        out_shape=jax.ShapeDtypeStruct((1, D), f32),
    )(x_operation, Wo0, _row(bo0), Wo1, _row(bo1))

    S = _SHIP_TILE
    n_act = N_SHIPS // S
    logits = pl.pallas_call(
        _actor_kernel,
        grid=(n_act,),
        in_specs=[
            pl.BlockSpec((S, 128), lambda i: (i, 0)),
            pl.BlockSpec((S * NUM_QUAY, 2), lambda i: (i, 0)),
            full((NUM_QUAY, D)),
            full((128, D)), full((1, D)), full((D, D)), full((1, D)),
            full((2, D)), full((1, D)), full((D, D)), full((1, D)),
            full((D, D)), full((D, D)), full((D, D)), full((1, D)),
            full((D, 1)), full((1, 1)),
        ],
        out_specs=pl.BlockSpec((S, NUM_QUAY), lambda i: (i, 0)),
        out_shape=jax.ShapeDtypeStruct((N_SHIPS, NUM_QUAY), f32),
    )(xg, ai, qterm, Wo0, _row(bo0), Wo1, _row(bo1), Wf0, _row(bf0),
      Wf1, _row(bf1), Wa0s, Wa0f, Wa1, _row(ba1), Wa2, _row(ba2))

    action, logprob, sv = pl.pallas_call(
        _final_kernel,
        out_shape=(jax.ShapeDtypeStruct((1, 1), jnp.int32),
                   jax.ShapeDtypeStruct((1, 1), f32),
                   jax.ShapeDtypeStruct((1, 1), f32)),
    )(logits, mask_t, qpool, opsum, Wc0q, Wc0o, _row(bc0), Wc1, _row(bc1),
      Wc2, _row(bc2))

    return (action.reshape(()), logprob.reshape(()), sv.reshape(()))


# speedup vs baseline: 1.8906x; 1.8906x over previous
import jax
import jax.numpy as jnp
from jax.experimental import pallas as pl

NUM_QUAY = 64
NUM_OPS = 50000
N_SHIPS = 1024

_POOL_TILE = 2000
_SHIP_TILE = 64


def _elu(x):
    return jnp.where(x > 0, x, jnp.exp(x) - 1.0)


def _dot(a, b):
    return jnp.dot(a, b, preferred_element_type=jnp.float32)


def _dot_hi(a, b):
    return jnp.dot(a, b, preferred_element_type=jnp.float32,
                   precision=jax.lax.Precision.HIGHEST)


def _quay_kernel(xq, Wq0, bq0, Wq1, bq1, Wa0q, ba0, qterm_ref, qpool_ref):
    h = _elu(_dot(xq[...], Wq0[...]) + bq0[...])
    h = _elu(_dot(h, Wq1[...]) + bq1[...])
    qterm_ref[...] = _dot(h, Wa0q[...]) + ba0[...]
    qpool_ref[...] = jnp.mean(h, axis=0, keepdims=True)


def _pool_kernel(xo, Wo0, bo0, Wo1, bo1, acc_ref):
    h = _elu(_dot(xo[...], Wo0[...]) + bo0[...])
    h = _elu(_dot(h, Wo1[...]) + bo1[...])
    part = jnp.sum(h, axis=0, keepdims=True)

    @pl.when(pl.program_id(0) == 0)
    def _init():
        acc_ref[...] = part

    @pl.when(pl.program_id(0) != 0)
    def _acc():
        acc_ref[...] += part


def _actor_kernel(xg, ai, qterm, Wo0, bo0, Wo1, bo1, Wf0, bf0, Wf1, bf1,
                  Wa0s, Wa0f, Wa1, ba1, Wa2, ba2, out_ref):
    S = xg.shape[0]
    B = S * NUM_QUAY
    h = _elu(_dot(xg[...], Wo0[...]) + bo0[...])
    h_ship = _elu(_dot(h, Wo1[...]) + bo1[...])
    s_term = _dot(h_ship, Wa0s[...])
    a1 = _elu(_dot(ai[...], Wf0[...]) + bf0[...])
    h_add = _elu(_dot(a1, Wf1[...]) + bf1[...])
    act = _dot(h_add, Wa0f[...])
    act = act.reshape(S, NUM_QUAY, 256) + s_term[:, None, :] + qterm[...][None, :, :]
    act = _elu(act).reshape(B, 256)
    h2 = _elu(_dot(act, Wa1[...]) + ba1[...])
    logit = _dot(h2, Wa2[...]) + ba2[...]
    out_ref[...] = logit.reshape(S, NUM_QUAY)


def _final_kernel(logits, mask_t, qpool, opool, Wc0q, Wc0o, bc0, Wc1, bc1,
                  Wc2, bc2, act_ref, lp_ref, sv_ref):
    lg = jnp.where(mask_t[...], logits[...], -jnp.inf)
    m = jnp.max(lg)
    ids = (jax.lax.broadcasted_iota(jnp.int32, lg.shape, 0) * NUM_QUAY
           + jax.lax.broadcasted_iota(jnp.int32, lg.shape, 1))
    amax = jnp.min(jnp.where(lg == m, ids, jnp.int32(2**31 - 1)))
    lse = m + jnp.log(jnp.sum(jnp.exp(lg - m)))
    act_ref[...] = amax.reshape(1, 1)
    lp_ref[...] = (m - lse).reshape(1, 1)
    hc = _elu(_dot(qpool[...], Wc0q[...]) + _dot(opool[...], Wc0o[...]) + bc0[...])
    hc = _elu(_dot(hc, Wc1[...]) + bc1[...])
    sv_ref[...] = _dot_hi(hc, Wc2[...]) + bc2[...]


def _row(x):
    return x.reshape(1, -1)


def kernel(x_quay, x_operation, mask, current_ops, added_info, Wq0, bq0, Wq1,
           bq1, Wo0, bo0, Wo1, bo1, Wf0, bf0, Wf1, bf1, Wa0, ba0, Wa1, ba1,
           Wa2, ba2, Wc0, bc0, Wc1, bc1, Wc2, bc2):
    f32 = jnp.float32
    D = 256
    Wa0q, Wa0s, Wa0f = Wa0[:D], Wa0[D:2 * D], Wa0[2 * D:]
    Wc0q, Wc0o = Wc0[:D], Wc0[D:]
    ai = added_info.reshape(N_SHIPS * NUM_QUAY, 2)
    mask_t = jnp.transpose(mask)
    xg = jnp.take(x_operation, current_ops, axis=0)

    full = lambda shape: pl.BlockSpec(shape, lambda *_: tuple(0 for _ in shape))

    qterm, qpool = pl.pallas_call(
        _quay_kernel,
        out_shape=(jax.ShapeDtypeStruct((NUM_QUAY, D), f32),
                   jax.ShapeDtypeStruct((1, D), f32)),
    )(x_quay, Wq0, _row(bq0), Wq1, _row(bq1), Wa0q, _row(ba0))

    n_pool = NUM_OPS // _POOL_TILE
    opsum = pl.pallas_call(
        _pool_kernel,
        grid=(n_pool,),
        in_specs=[
            pl.BlockSpec((_POOL_TILE, 128), lambda i: (i, 0)),
            full((128, D)), full((1, D)), full((D, D)), full((1, D)),
        ],
        out_specs=pl.BlockSpec((1, D), lambda i: (0, 0)),
        out_shape=jax.ShapeDtypeStruct((1, D), f32),
    )(x_operation, Wo0, _row(bo0), Wo1, _row(bo1))

    S = _SHIP_TILE
    n_act = N_SHIPS // S
    logits = pl.pallas_call(
        _actor_kernel,
        grid=(n_act,),
        in_specs=[
            pl.BlockSpec((S, 128), lambda i: (i, 0)),
            pl.BlockSpec((S * NUM_QUAY, 2), lambda i: (i, 0)),
            full((NUM_QUAY, D)),
            full((128, D)), full((1, D)), full((D, D)), full((1, D)),
            full((2, D)), full((1, D)), full((D, D)), full((1, D)),
            full((D, D)), full((D, D)), full((D, D)), full((1, D)),
            full((D, 1)), full((1, 1)),
        ],
        out_specs=pl.BlockSpec((S, NUM_QUAY), lambda i: (i, 0)),
        out_shape=jax.ShapeDtypeStruct((N_SHIPS, NUM_QUAY), f32),
    )(xg, ai, qterm, Wo0, _row(bo0), Wo1, _row(bo1), Wf0, _row(bf0),
      Wf1, _row(bf1), Wa0s, Wa0f, Wa1, _row(ba1), Wa2, _row(ba2))

    action, logprob, sv = pl.pallas_call(
        _final_kernel,
        out_shape=(jax.ShapeDtypeStruct((1, 1), jnp.int32),
                   jax.ShapeDtypeStruct((1, 1), f32),
                   jax.ShapeDtypeStruct((1, 1), f32)),
    )(logits, mask_t, qpool, opsum / jnp.float32(NUM_OPS), Wc0q, Wc0o,
      _row(bc0), Wc1, _row(bc1), Wc2, _row(bc2))

    return (action.reshape(()), logprob.reshape(()), sv.reshape(()))


# speedup vs baseline: 1.9301x; 1.0209x over previous
import jax
import jax.numpy as jnp
from jax.experimental import pallas as pl

NUM_QUAY = 64
NUM_OPS = 50000
N_SHIPS = 1024

_POOL_TILE = 5000
_SHIP_TILE = 128


def _elu(x):
    return jnp.where(x > 0, x, jnp.exp(x) - 1.0)


def _dot(a, b):
    return jnp.dot(a, b, preferred_element_type=jnp.float32)


def _dot_hi(a, b):
    return jnp.dot(a, b, preferred_element_type=jnp.float32,
                   precision=jax.lax.Precision.HIGHEST)


def _quay_kernel(xq, Wq0, bq0, Wq1, bq1, Wa0q, ba0, qterm_ref, qpool_ref):
    h = _elu(_dot(xq[...], Wq0[...]) + bq0[...])
    h = _elu(_dot(h, Wq1[...]) + bq1[...])
    qterm_ref[...] = _dot(h, Wa0q[...]) + ba0[...]
    qpool_ref[...] = jnp.mean(h, axis=0, keepdims=True)


def _pool_kernel(xo, Wo0, bo0, Wo1, bo1, acc_ref):
    h = _elu(_dot(xo[...], Wo0[...]) + bo0[...])
    h = _elu(_dot(h, Wo1[...]) + bo1[...])
    part = jnp.sum(h, axis=0, keepdims=True)

    @pl.when(pl.program_id(0) == 0)
    def _init():
        acc_ref[...] = part

    @pl.when(pl.program_id(0) != 0)
    def _acc():
        acc_ref[...] += part


def _actor_kernel(xg, ai, qterm, Wo0, bo0, Wo1, bo1, Wf0, bf0, Wf1, bf1,
                  Wa0s, Wa0f, Wa1, ba1, Wa2, ba2, out_ref):
    S = xg.shape[0]
    B = S * NUM_QUAY
    h = _elu(_dot(xg[...], Wo0[...]) + bo0[...])
    h_ship = _elu(_dot(h, Wo1[...]) + bo1[...])
    s_term = _dot(h_ship, Wa0s[...])
    a1 = _elu(_dot(ai[...], Wf0[...]) + bf0[...])
    h_add = _elu(_dot(a1, Wf1[...]) + bf1[...])
    act = _dot(h_add, Wa0f[...])
    act = act.reshape(S, NUM_QUAY, 256) + s_term[:, None, :] + qterm[...][None, :, :]
    act = _elu(act).reshape(B, 256)
    h2 = _elu(_dot(act, Wa1[...]) + ba1[...])
    logit = _dot(h2, Wa2[...]) + ba2[...]
    out_ref[...] = logit.reshape(S, NUM_QUAY)


def _final_kernel(logits, mask_t, qpool, opool, Wc0q, Wc0o, bc0, Wc1, bc1,
                  Wc2, bc2, act_ref, lp_ref, sv_ref):
    lg = jnp.where(mask_t[...], logits[...], -jnp.inf)
    m = jnp.max(lg)
    ids = (jax.lax.broadcasted_iota(jnp.int32, lg.shape, 0) * NUM_QUAY
           + jax.lax.broadcasted_iota(jnp.int32, lg.shape, 1))
    amax = jnp.min(jnp.where(lg == m, ids, jnp.int32(2**31 - 1)))
    lse = m + jnp.log(jnp.sum(jnp.exp(lg - m)))
    act_ref[...] = amax.reshape(1, 1)
    lp_ref[...] = (m - lse).reshape(1, 1)
    hc = _elu(_dot(qpool[...], Wc0q[...]) + _dot(opool[...], Wc0o[...]) + bc0[...])
    hc = _elu(_dot(hc, Wc1[...]) + bc1[...])
    sv_ref[...] = _dot_hi(hc, Wc2[...]) + bc2[...]


def _row(x):
    return x.reshape(1, -1)


def kernel(x_quay, x_operation, mask, current_ops, added_info, Wq0, bq0, Wq1,
           bq1, Wo0, bo0, Wo1, bo1, Wf0, bf0, Wf1, bf1, Wa0, ba0, Wa1, ba1,
           Wa2, ba2, Wc0, bc0, Wc1, bc1, Wc2, bc2):
    f32 = jnp.float32
    D = 256
    Wa0q, Wa0s, Wa0f = Wa0[:D], Wa0[D:2 * D], Wa0[2 * D:]
    Wc0q, Wc0o = Wc0[:D], Wc0[D:]
    ai = added_info.reshape(N_SHIPS * NUM_QUAY, 2)
    mask_t = jnp.transpose(mask)
    xg = jnp.take(x_operation, current_ops, axis=0)

    full = lambda shape: pl.BlockSpec(shape, lambda *_: tuple(0 for _ in shape))

    qterm, qpool = pl.pallas_call(
        _quay_kernel,
        out_shape=(jax.ShapeDtypeStruct((NUM_QUAY, D), f32),
                   jax.ShapeDtypeStruct((1, D), f32)),
    )(x_quay, Wq0, _row(bq0), Wq1, _row(bq1), Wa0q, _row(ba0))

    n_pool = NUM_OPS // _POOL_TILE
    opsum = pl.pallas_call(
        _pool_kernel,
        grid=(n_pool,),
        in_specs=[
            pl.BlockSpec((_POOL_TILE, 128), lambda i: (i, 0)),
            full((128, D)), full((1, D)), full((D, D)), full((1, D)),
        ],
        out_specs=pl.BlockSpec((1, D), lambda i: (0, 0)),
        out_shape=jax.ShapeDtypeStruct((1, D), f32),
    )(x_operation, Wo0, _row(bo0), Wo1, _row(bo1))

    S = _SHIP_TILE
    n_act = N_SHIPS // S
    logits = pl.pallas_call(
        _actor_kernel,
        grid=(n_act,),
        in_specs=[
            pl.BlockSpec((S, 128), lambda i: (i, 0)),
            pl.BlockSpec((S * NUM_QUAY, 2), lambda i: (i, 0)),
            full((NUM_QUAY, D)),
            full((128, D)), full((1, D)), full((D, D)), full((1, D)),
            full((2, D)), full((1, D)), full((D, D)), full((1, D)),
            full((D, D)), full((D, D)), full((D, D)), full((1, D)),
            full((D, 1)), full((1, 1)),
        ],
        out_specs=pl.BlockSpec((S, NUM_QUAY), lambda i: (i, 0)),
        out_shape=jax.ShapeDtypeStruct((N_SHIPS, NUM_QUAY), f32),
    )(xg, ai, qterm, Wo0, _row(bo0), Wo1, _row(bo1), Wf0, _row(bf0),
      Wf1, _row(bf1), Wa0s, Wa0f, Wa1, _row(ba1), Wa2, _row(ba2))

    action, logprob, sv = pl.pallas_call(
        _final_kernel,
        out_shape=(jax.ShapeDtypeStruct((1, 1), jnp.int32),
                   jax.ShapeDtypeStruct((1, 1), f32),
                   jax.ShapeDtypeStruct((1, 1), f32)),
    )(logits, mask_t, qpool, opsum / jnp.float32(NUM_OPS), Wc0q, Wc0o,
      _row(bc0), Wc1, _row(bc1), Wc2, _row(bc2))

    return (action.reshape(()), logprob.reshape(()), sv.reshape(()))


# speedup vs baseline: 1.9468x; 1.0086x over previous
import jax
import jax.numpy as jnp
from jax.experimental import pallas as pl

NUM_QUAY = 64
NUM_OPS = 50000
N_SHIPS = 1024

_POOL_TILE = 10000
_SHIP_TILE = 256


def _elu(x):
    return jnp.where(x > 0, x, jnp.exp(x) - 1.0)


def _dot(a, b):
    return jnp.dot(a, b, preferred_element_type=jnp.float32)


def _dot_hi(a, b):
    return jnp.dot(a, b, preferred_element_type=jnp.float32,
                   precision=jax.lax.Precision.HIGHEST)


def _quay_kernel(xq, Wq0, bq0, Wq1, bq1, Wa0q, ba0, qterm_ref, qpool_ref):
    h = _elu(_dot(xq[...], Wq0[...]) + bq0[...])
    h = _elu(_dot(h, Wq1[...]) + bq1[...])
    qterm_ref[...] = _dot(h, Wa0q[...]) + ba0[...]
    qpool_ref[...] = jnp.mean(h, axis=0, keepdims=True)


def _pool_kernel(xo, Wo0, bo0, Wo1, bo1, acc_ref):
    h = _elu(_dot(xo[...], Wo0[...]) + bo0[...])
    h = _elu(_dot(h, Wo1[...]) + bo1[...])
    part = jnp.sum(h, axis=0, keepdims=True)

    @pl.when(pl.program_id(0) == 0)
    def _init():
        acc_ref[...] = part

    @pl.when(pl.program_id(0) != 0)
    def _acc():
        acc_ref[...] += part


def _actor_kernel(xg, ai, qterm, Wo0, bo0, Wo1, bo1, Wf0, bf0, Wf1, bf1,
                  Wa0s, Wa0f, Wa1, ba1, Wa2, ba2, out_ref):
    S = xg.shape[0]
    B = S * NUM_QUAY
    h = _elu(_dot(xg[...], Wo0[...]) + bo0[...])
    h_ship = _elu(_dot(h, Wo1[...]) + bo1[...])
    s_term = _dot(h_ship, Wa0s[...])
    a1 = _elu(_dot(ai[...], Wf0[...]) + bf0[...])
    h_add = _elu(_dot(a1, Wf1[...]) + bf1[...])
    act = _dot(h_add, Wa0f[...])
    act = act.reshape(S, NUM_QUAY, 256) + s_term[:, None, :] + qterm[...][None, :, :]
    act = _elu(act).reshape(B, 256)
    h2 = _elu(_dot(act, Wa1[...]) + ba1[...])
    logit = _dot(h2, Wa2[...]) + ba2[...]
    out_ref[...] = logit.reshape(S, NUM_QUAY)


def _final_kernel(logits, mask_t, qpool, opool, Wc0q, Wc0o, bc0, Wc1, bc1,
                  Wc2, bc2, act_ref, lp_ref, sv_ref):
    lg = jnp.where(mask_t[...], logits[...], -jnp.inf)
    m = jnp.max(lg)
    ids = (jax.lax.broadcasted_iota(jnp.int32, lg.shape, 0) * NUM_QUAY
           + jax.lax.broadcasted_iota(jnp.int32, lg.shape, 1))
    amax = jnp.min(jnp.where(lg == m, ids, jnp.int32(2**31 - 1)))
    lse = m + jnp.log(jnp.sum(jnp.exp(lg - m)))
    act_ref[...] = amax.reshape(1, 1)
    lp_ref[...] = (m - lse).reshape(1, 1)
    hc = _elu(_dot(qpool[...], Wc0q[...]) + _dot(opool[...], Wc0o[...]) + bc0[...])
    hc = _elu(_dot(hc, Wc1[...]) + bc1[...])
    sv_ref[...] = _dot_hi(hc, Wc2[...]) + bc2[...]


def _row(x):
    return x.reshape(1, -1)


def kernel(x_quay, x_operation, mask, current_ops, added_info, Wq0, bq0, Wq1,
           bq1, Wo0, bo0, Wo1, bo1, Wf0, bf0, Wf1, bf1, Wa0, ba0, Wa1, ba1,
           Wa2, ba2, Wc0, bc0, Wc1, bc1, Wc2, bc2):
    f32 = jnp.float32
    D = 256
    Wa0q, Wa0s, Wa0f = Wa0[:D], Wa0[D:2 * D], Wa0[2 * D:]
    Wc0q, Wc0o = Wc0[:D], Wc0[D:]
    ai = added_info.reshape(N_SHIPS * NUM_QUAY, 2)
    mask_t = jnp.transpose(mask)
    xg = jnp.take(x_operation, current_ops, axis=0)

    full = lambda shape: pl.BlockSpec(shape, lambda *_: tuple(0 for _ in shape))

    qterm, qpool = pl.pallas_call(
        _quay_kernel,
        out_shape=(jax.ShapeDtypeStruct((NUM_QUAY, D), f32),
                   jax.ShapeDtypeStruct((1, D), f32)),
    )(x_quay, Wq0, _row(bq0), Wq1, _row(bq1), Wa0q, _row(ba0))

    n_pool = NUM_OPS // _POOL_TILE
    opsum = pl.pallas_call(
        _pool_kernel,
        grid=(n_pool,),
        in_specs=[
            pl.BlockSpec((_POOL_TILE, 128), lambda i: (i, 0)),
            full((128, D)), full((1, D)), full((D, D)), full((1, D)),
        ],
        out_specs=pl.BlockSpec((1, D), lambda i: (0, 0)),
        out_shape=jax.ShapeDtypeStruct((1, D), f32),
    )(x_operation, Wo0, _row(bo0), Wo1, _row(bo1))

    S = _SHIP_TILE
    n_act = N_SHIPS // S
    logits = pl.pallas_call(
        _actor_kernel,
        grid=(n_act,),
        in_specs=[
            pl.BlockSpec((S, 128), lambda i: (i, 0)),
            pl.BlockSpec((S * NUM_QUAY, 2), lambda i: (i, 0)),
            full((NUM_QUAY, D)),
            full((128, D)), full((1, D)), full((D, D)), full((1, D)),
            full((2, D)), full((1, D)), full((D, D)), full((1, D)),
            full((D, D)), full((D, D)), full((D, D)), full((1, D)),
            full((D, 1)), full((1, 1)),
        ],
        out_specs=pl.BlockSpec((S, NUM_QUAY), lambda i: (i, 0)),
        out_shape=jax.ShapeDtypeStruct((N_SHIPS, NUM_QUAY), f32),
    )(xg, ai, qterm, Wo0, _row(bo0), Wo1, _row(bo1), Wf0, _row(bf0),
      Wf1, _row(bf1), Wa0s, Wa0f, Wa1, _row(ba1), Wa2, _row(ba2))

    action, logprob, sv = pl.pallas_call(
        _final_kernel,
        out_shape=(jax.ShapeDtypeStruct((1, 1), jnp.int32),
                   jax.ShapeDtypeStruct((1, 1), f32),
                   jax.ShapeDtypeStruct((1, 1), f32)),
    )(logits, mask_t, qpool, opsum / jnp.float32(NUM_OPS), Wc0q, Wc0o,
      _row(bc0), Wc1, _row(bc1), Wc2, _row(bc2))

    return (action.reshape(()), logprob.reshape(()), sv.reshape(()))


# speedup vs baseline: 1.9478x; 1.0005x over previous
import jax
import jax.numpy as jnp
from jax.experimental import pallas as pl

NUM_QUAY = 64
NUM_OPS = 50000
N_SHIPS = 1024

_POOL_TILE = 5000
_SHIP_TILE = 256


def _elu(x):
    return jnp.where(x > 0, x, jnp.exp(x) - 1.0)


def _dot(a, b):
    return jnp.dot(a, b, preferred_element_type=jnp.float32)


def _dot_hi(a, b):
    return jnp.dot(a, b, preferred_element_type=jnp.float32,
                   precision=jax.lax.Precision.HIGHEST)


def _quay_kernel(xq, Wq0, bq0, Wq1, bq1, Wa0q, ba0, qterm_ref, qpool_ref):
    h = _elu(_dot(xq[...], Wq0[...]) + bq0[...])
    h = _elu(_dot(h, Wq1[...]) + bq1[...])
    qterm_ref[...] = _dot(h, Wa0q[...]) + ba0[...]
    qpool_ref[...] = jnp.mean(h, axis=0, keepdims=True)


def _pool_kernel(xo, Wo0, bo0, Wo1, bo1, acc_ref):
    h = _elu(_dot(xo[...], Wo0[...]) + bo0[...])
    h = _elu(_dot(h, Wo1[...]) + bo1[...])
    part = jnp.sum(h, axis=0, keepdims=True)

    @pl.when(pl.program_id(0) == 0)
    def _init():
        acc_ref[...] = part

    @pl.when(pl.program_id(0) != 0)
    def _acc():
        acc_ref[...] += part


def _actor_kernel(xg, ai, qterm, Wo0, bo0, Wo1, bo1, Wf0, bf0, Wf1, bf1,
                  Wa0s, Wa0f, Wa1, ba1, Wa2, ba2, out_ref):
    S = xg.shape[0]
    B = S * NUM_QUAY
    h = _elu(_dot(xg[...], Wo0[...]) + bo0[...])
    h_ship = _elu(_dot(h, Wo1[...]) + bo1[...])
    s_term = _dot(h_ship, Wa0s[...])
    a1 = _elu(_dot(ai[...], Wf0[...]) + bf0[...])
    h_add = _elu(_dot(a1, Wf1[...]) + bf1[...])
    act = _dot(h_add, Wa0f[...])
    act = act.reshape(S, NUM_QUAY, 256) + s_term[:, None, :] + qterm[...][None, :, :]
    act = _elu(act).reshape(B, 256)
    h2 = _elu(_dot(act, Wa1[...]) + ba1[...])
    logit = _dot(h2, Wa2[...]) + ba2[...]
    out_ref[...] = logit.reshape(S, NUM_QUAY)


def _final_kernel(logits, mask_t, qpool, opool, Wc0q, Wc0o, bc0, Wc1, bc1,
                  Wc2, bc2, act_ref, lp_ref, sv_ref):
    lg = jnp.where(mask_t[...], logits[...], -jnp.inf)
    m = jnp.max(lg)
    ids = (jax.lax.broadcasted_iota(jnp.int32, lg.shape, 0) * NUM_QUAY
           + jax.lax.broadcasted_iota(jnp.int32, lg.shape, 1))
    amax = jnp.min(jnp.where(lg == m, ids, jnp.int32(2**31 - 1)))
    lse = m + jnp.log(jnp.sum(jnp.exp(lg - m)))
    act_ref[...] = amax.reshape(1, 1)
    lp_ref[...] = (m - lse).reshape(1, 1)
    hc = _elu(_dot(qpool[...], Wc0q[...]) + _dot(opool[...], Wc0o[...]) + bc0[...])
    hc = _elu(_dot(hc, Wc1[...]) + bc1[...])
    sv_ref[...] = _dot_hi(hc, Wc2[...]) + bc2[...]


def _row(x):
    return x.reshape(1, -1)


def kernel(x_quay, x_operation, mask, current_ops, added_info, Wq0, bq0, Wq1,
           bq1, Wo0, bo0, Wo1, bo1, Wf0, bf0, Wf1, bf1, Wa0, ba0, Wa1, ba1,
           Wa2, ba2, Wc0, bc0, Wc1, bc1, Wc2, bc2):
    f32 = jnp.float32
    D = 256
    Wa0q, Wa0s, Wa0f = Wa0[:D], Wa0[D:2 * D], Wa0[2 * D:]
    Wc0q, Wc0o = Wc0[:D], Wc0[D:]
    ai = added_info.reshape(N_SHIPS * NUM_QUAY, 2)
    mask_t = jnp.transpose(mask)
    xg = jnp.take(x_operation, current_ops, axis=0)

    full = lambda shape: pl.BlockSpec(shape, lambda *_: tuple(0 for _ in shape))

    qterm, qpool = pl.pallas_call(
        _quay_kernel,
        out_shape=(jax.ShapeDtypeStruct((NUM_QUAY, D), f32),
                   jax.ShapeDtypeStruct((1, D), f32)),
    )(x_quay, Wq0, _row(bq0), Wq1, _row(bq1), Wa0q, _row(ba0))

    n_pool = NUM_OPS // _POOL_TILE
    opsum = pl.pallas_call(
        _pool_kernel,
        grid=(n_pool,),
        in_specs=[
            pl.BlockSpec((_POOL_TILE, 128), lambda i: (i, 0)),
            full((128, D)), full((1, D)), full((D, D)), full((1, D)),
        ],
        out_specs=pl.BlockSpec((1, D), lambda i: (0, 0)),
        out_shape=jax.ShapeDtypeStruct((1, D), f32),
    )(x_operation, Wo0, _row(bo0), Wo1, _row(bo1))

    S = _SHIP_TILE
    n_act = N_SHIPS // S
    logits = pl.pallas_call(
        _actor_kernel,
        grid=(n_act,),
        in_specs=[
            pl.BlockSpec((S, 128), lambda i: (i, 0)),
            pl.BlockSpec((S * NUM_QUAY, 2), lambda i: (i, 0)),
            full((NUM_QUAY, D)),
            full((128, D)), full((1, D)), full((D, D)), full((1, D)),
            full((2, D)), full((1, D)), full((D, D)), full((1, D)),
            full((D, D)), full((D, D)), full((D, D)), full((1, D)),
            full((D, 1)), full((1, 1)),
        ],
        out_specs=pl.BlockSpec((S, NUM_QUAY), lambda i: (i, 0)),
        out_shape=jax.ShapeDtypeStruct((N_SHIPS, NUM_QUAY), f32),
    )(xg, ai, qterm, Wo0, _row(bo0), Wo1, _row(bo1), Wf0, _row(bf0),
      Wf1, _row(bf1), Wa0s, Wa0f, Wa1, _row(ba1), Wa2, _row(ba2))

    action, logprob, sv = pl.pallas_call(
        _final_kernel,
        out_shape=(jax.ShapeDtypeStruct((1, 1), jnp.int32),
                   jax.ShapeDtypeStruct((1, 1), f32),
                   jax.ShapeDtypeStruct((1, 1), f32)),
    )(logits, mask_t, qpool, opsum / jnp.float32(NUM_OPS), Wc0q, Wc0o,
      _row(bc0), Wc1, _row(bc1), Wc2, _row(bc2))

    return (action.reshape(()), logprob.reshape(()), sv.reshape(()))
